# baseline (speedup 1.0000x reference)
import functools

import jax
import jax.numpy as jnp
from jax import lax
from jax.experimental import pallas as pl
from jax.experimental.pallas import tpu as pltpu
from jax.experimental.pallas import tpu_sc as plsc

NUM_EMB = 1000000
LATENT = 32
HIDDEN = 128

NC = 2
NS = 16
NW = NC * NS

CHUNK = 320
NBUF = 2


def _sc_gather(table, idx, n_tokens):
    bpw = n_tokens // NW
    nchunk = bpw // CHUNK
    grp = CHUNK // 4
    orpc = CHUNK // 4

    mesh = plsc.VectorSubcoreMesh(core_axis_name="c", subcore_axis_name="s")

    @functools.partial(
        pl.kernel,
        mesh=mesh,
        compiler_params=pltpu.CompilerParams(use_tc_tiling_on_sc=False),
        out_type=jax.ShapeDtypeStruct((n_tokens * LATENT // HIDDEN, HIDDEN), jnp.float32),
        scratch_types=[
            pltpu.VMEM((bpw,), jnp.int32),
            [[pltpu.VMEM((grp, LATENT), jnp.float32) for _ in range(4)]
             for _ in range(NBUF)],
            [pltpu.SemaphoreType.DMA for _ in range(NBUF)],
        ],
    )
    def gather_kernel(table_hbm, idx_hbm, out_hbm, idx_v, bufs, sems):
        wid = lax.axis_index("s") * NC + lax.axis_index("c")
        base = wid * bpw
        out_base = wid * (bpw * LATENT // HIDDEN)
        pltpu.sync_copy(idx_hbm.at[pl.ds(base, bpw)], idx_v)

        def issue(j, b):
            for a in range(4):
                pltpu.async_copy(
                    table_hbm.at[idx_v.at[pl.ds(j * CHUNK + a * grp, grp)]],
                    bufs[b][a],
                    sems[b],
                )

        def drain(j, b):
            for a in range(4):
                pltpu.make_async_copy(
                    table_hbm.at[idx_v.at[pl.ds(j * CHUNK + a * grp, grp)]],
                    bufs[b][a],
                    sems[b],
                ).wait()
            for a in range(4):
                pltpu.sync_copy(
                    bufs[b][a],
                    out_hbm.at[
                        pl.ds(out_base + j * orpc, orpc),
                        pl.ds(a * LATENT, LATENT),
                    ],
                )

        for b in range(NBUF):
            issue(b, b)

        def body(g, carry):
            for b in range(NBUF):
                j = g * NBUF + b
                drain(j, b)
                issue(j + NBUF, b)
            return carry

        lax.fori_loop(0, nchunk // NBUF - 1, body, 0)

        for b in range(NBUF):
            drain(nchunk - NBUF + b, b)

    return gather_kernel(table, idx)


def _tc_project(h2, w4, batch, seq):
    blk_rows = 1600
    blk_batch = blk_rows * 4 // seq
    grid = batch // blk_batch

    n_chunks = blk_rows * 4 // CHUNK
    grp = CHUNK // 4

    def mm_kernel(h_ref, w_ref, o_ref):
        m = jnp.dot(h_ref[...], w_ref[...], preferred_element_type=jnp.float32)
        m = m.reshape(n_chunks, grp, 4, HIDDEN).transpose(0, 2, 1, 3)
        o_ref[...] = m.reshape(blk_batch, seq, HIDDEN)

    return pl.pallas_call(
        mm_kernel,
        grid=(grid,),
        in_specs=[
            pl.BlockSpec((blk_rows, HIDDEN), lambda i: (i, 0)),
            pl.BlockSpec((HIDDEN, 4 * HIDDEN), lambda i: (0, 0)),
        ],
        out_specs=pl.BlockSpec((blk_batch, seq, HIDDEN), lambda i: (i, 0, 0)),
        out_shape=jax.ShapeDtypeStruct((batch, seq, HIDDEN), jnp.float32),
    )(h2, w4)


def kernel(x, emb1_weight, emb2_weight):
    batch, seq = x.shape
    n_tokens = batch * seq

    idx = x.reshape(-1)
    h2 = _sc_gather(emb1_weight, idx, n_tokens)

    wt = emb2_weight.T
    eye4 = jnp.eye(4, dtype=jnp.float32)
    w4 = (eye4[:, None, :, None] * wt[None, :, None, :]).reshape(
        4 * LATENT, 4 * HIDDEN
    )

    return _tc_project(h2, w4, batch, seq)


# speedup vs baseline: 1.0178x; 1.0178x over previous
import functools

import jax
import jax.numpy as jnp
from jax import lax
from jax.experimental import pallas as pl
from jax.experimental.pallas import tpu as pltpu
from jax.experimental.pallas import tpu_sc as plsc

NUM_EMB = 1000000
LATENT = 32
HIDDEN = 128

NC = 2
NS = 16
NW = NC * NS

NBUF = 2


def _sc_gather(table, x2d, batch, seq):
    n_tokens = batch * seq
    rpw = batch // NW
    nchunk = rpw // 4

    mesh = plsc.VectorSubcoreMesh(core_axis_name="c", subcore_axis_name="s")

    @functools.partial(
        pl.kernel,
        mesh=mesh,
        compiler_params=pltpu.CompilerParams(use_tc_tiling_on_sc=False),
        out_type=jax.ShapeDtypeStruct((n_tokens * LATENT // HIDDEN, HIDDEN), jnp.float32),
        scratch_types=[
            pltpu.VMEM((rpw, seq), jnp.int32),
            [[pltpu.VMEM((seq, LATENT), jnp.float32) for _ in range(4)]
             for _ in range(NBUF)],
            [pltpu.SemaphoreType.DMA for _ in range(NBUF)],
        ],
    )
    def gather_kernel(table_hbm, x_hbm, out_hbm, idx_v, bufs, sems):
        wid = lax.axis_index("s") * NC + lax.axis_index("c")
        out_base = wid * (rpw * seq * LATENT // HIDDEN)
        pltpu.sync_copy(x_hbm.at[pl.ds(wid * rpw, rpw), :], idx_v)

        def issue(j, b):
            for a in range(4):
                pltpu.async_copy(
                    table_hbm.at[idx_v.at[4 * j + a]], bufs[b][a], sems[b]
                )

        def drain(j, b):
            for a in range(4):
                pltpu.make_async_copy(
                    table_hbm.at[idx_v.at[4 * j + a]], bufs[b][a], sems[b]
                ).wait()
            for a in range(4):
                pltpu.sync_copy(
                    bufs[b][a],
                    out_hbm.at[
                        pl.ds(out_base + j * seq, seq),
                        pl.ds(a * LATENT, LATENT),
                    ],
                )

        for b in range(NBUF):
            issue(b, b)

        def body(g, carry):
            for b in range(NBUF):
                j = g * NBUF + b
                drain(j, b)
                issue(j + NBUF, b)
            return carry

        lax.fori_loop(0, nchunk // NBUF - 1, body, 0)

        for b in range(NBUF):
            drain(nchunk - NBUF + b, b)

    return gather_kernel(table, x2d)


def _tc_project(h2, w4, batch, seq):
    blk_rows = 1600
    blk_batch = blk_rows * 4 // seq
    grid = batch // blk_batch
    n_chunks = blk_batch // 4

    def mm_kernel(h_ref, w_ref, o_ref):
        m = jnp.dot(h_ref[...], w_ref[...], preferred_element_type=jnp.float32)
        m = m.reshape(n_chunks, seq, 4, HIDDEN).transpose(0, 2, 1, 3)
        o_ref[...] = m.reshape(blk_batch, seq, HIDDEN)

    return pl.pallas_call(
        mm_kernel,
        grid=(grid,),
        in_specs=[
            pl.BlockSpec((blk_rows, HIDDEN), lambda i: (i, 0)),
            pl.BlockSpec((HIDDEN, 4 * HIDDEN), lambda i: (0, 0)),
        ],
        out_specs=pl.BlockSpec((blk_batch, seq, HIDDEN), lambda i: (i, 0, 0)),
        out_shape=jax.ShapeDtypeStruct((batch, seq, HIDDEN), jnp.float32),
    )(h2, w4)


def kernel(x, emb1_weight, emb2_weight):
    batch, seq = x.shape

    h2 = _sc_gather(emb1_weight, x, batch, seq)

    wt = emb2_weight.T
    eye4 = jnp.eye(4, dtype=jnp.float32)
    w4 = (eye4[:, None, :, None] * wt[None, :, None, :]).reshape(
        4 * LATENT, 4 * HIDDEN
    )

    return _tc_project(h2, w4, batch, seq)
